# Initial kernel scaffold; baseline (speedup 1.0000x reference)
import jax
import jax.numpy as jnp
from jax.experimental import pallas as pl


def kernel(x, edge_index, batch, W1, b1, g1, be1, W2, b2, g2, be2, W3, b3, W4, b4):
    raise NotImplementedError("write your pallas kernel here")


# speedup vs baseline: 16.1451x; 16.1451x over previous
import functools

import jax
import jax.numpy as jnp
from jax import lax
from jax.experimental import pallas as pl
from jax.experimental.pallas import tpu as pltpu
from jax.experimental.pallas import tpu_sc as plsc

N_NODES = 10000
N_EDGES = 320000
IN_CH = 128
H1 = 64
H2 = 32
MLP1 = 16
NUM_GRAPHS = 64

NC = 2
NS = 16
NW = NC * NS
C = 128
K = 80
E_PAD = NW * K * C
R = 10112
STRIPE = R // NS
DUMMY = N_NODES

_mesh = plsc.VectorSubcoreMesh(core_axis_name="c", subcore_axis_name="s")
_sc_params = pltpu.CompilerParams(use_tc_tiling_on_sc=False)


def _worker_id():
    return lax.axis_index("c") * NS + lax.axis_index("s")


def _deg_body(col3, ones_hbm, zeros_hbm, out, idx_c, ones_v, acc):
    cid = lax.axis_index("c")
    sid = lax.axis_index("s")
    wid = _worker_id()
    pltpu.sync_copy(col3.at[wid], idx_c)
    pltpu.sync_copy(ones_hbm, ones_v)
    pltpu.sync_copy(zeros_hbm.at[pl.ds(sid * STRIPE, STRIPE)],
                    acc.at[pl.ds(sid * STRIPE, STRIPE)])
    plsc.subcore_barrier()

    def body(j, carry):
        pltpu.sync_copy(ones_v, acc.at[idx_c.at[j]], add=True)
        return carry

    lax.fori_loop(0, K, body, 0)
    plsc.subcore_barrier()
    pltpu.sync_copy(acc.at[pl.ds(sid * STRIPE, STRIPE)],
                    out.at[cid, pl.ds(sid * STRIPE, STRIPE)])


_deg_kernel = pl.kernel(
    _deg_body,
    out_type=jax.ShapeDtypeStruct((NC, R, 8), jnp.float32),
    mesh=_mesh,
    scratch_types=[
        pltpu.VMEM((K, C), jnp.int32),
        pltpu.VMEM((C, 8), jnp.float32),
        pltpu.VMEM_SHARED((R, 8), jnp.float32),
    ],
    compiler_params=_sc_params,
)


def _agg_body(row3, col3, xs, zeros_hbm, out, idx_r, idx_c, buf, acc, sem):
    cid = lax.axis_index("c")
    sid = lax.axis_index("s")
    wid = _worker_id()
    pltpu.sync_copy(row3.at[wid], idx_r)
    pltpu.sync_copy(col3.at[wid], idx_c)
    pltpu.sync_copy(zeros_hbm.at[pl.ds(sid * STRIPE, STRIPE)],
                    acc.at[pl.ds(sid * STRIPE, STRIPE)])
    plsc.subcore_barrier()

    def body(j, carry):
        pltpu.async_copy(xs.at[idx_r.at[j]], buf, sem).wait()
        pltpu.sync_copy(buf, acc.at[idx_c.at[j]], add=True)
        return carry

    lax.fori_loop(0, K, body, 0)
    plsc.subcore_barrier()
    pltpu.sync_copy(acc.at[pl.ds(sid * STRIPE, STRIPE)],
                    out.at[cid, pl.ds(sid * STRIPE, STRIPE)])


def _make_agg(d):
    return pl.kernel(
        _agg_body,
        out_type=jax.ShapeDtypeStruct((NC, R, d), jnp.float32),
        mesh=_mesh,
        scratch_types=[
            pltpu.VMEM((K, C), jnp.int32),
            pltpu.VMEM((K, C), jnp.int32),
            pltpu.VMEM((C, d), jnp.float32),
            pltpu.VMEM_SHARED((R, d), jnp.float32),
            pltpu.SemaphoreType.DMA,
        ],
        compiler_params=_sc_params,
    )


_agg64 = _make_agg(H1)
_agg32 = _make_agg(H2)


def _leaky(v):
    return jnp.where(v >= 0, v, 0.01 * v)


def _bn(h, g, be):
    mu = jnp.mean(h, axis=0, keepdims=True)
    var = jnp.mean((h - mu) * (h - mu), axis=0, keepdims=True)
    return (h - mu) * lax.rsqrt(var + 1e-5) * g + be


def _tc_a_body(degp_ref, x_ref, w1_ref, dinv_ref, xw1_ref, xs1_ref):
    deg = degp_ref[0, :N_NODES, 0:1] + degp_ref[1, :N_NODES, 0:1] + 1.0
    dinv = lax.rsqrt(deg)
    xw = jnp.dot(x_ref[...], w1_ref[...], preferred_element_type=jnp.float32)
    dinv_ref[...] = dinv
    xw1_ref[...] = xw
    xs1_ref[...] = xw * dinv


def _tc_b_body(sp_ref, xw1_ref, dinv_ref, b1_ref, g1_ref, be1_ref, w2_ref,
               xw2_ref, xs2_ref):
    s = sp_ref[0, :N_NODES, :] + sp_ref[1, :N_NODES, :]
    dv = dinv_ref[...]
    xw1 = xw1_ref[...]
    h = dv * s + (dv * dv) * xw1 + b1_ref[...]
    h = _bn(_leaky(h), g1_ref[...], be1_ref[...])
    xw2 = jnp.dot(h, w2_ref[...], preferred_element_type=jnp.float32)
    xw2_ref[...] = xw2
    xs2_ref[...] = xw2 * dv


def _tc_c_body(sp_ref, xw2_ref, dinv_ref, b2_ref, g2_ref, be2_ref, batch_ref,
               w3_ref, b3_ref, w4_ref, b4_ref, out_ref):
    s = sp_ref[0, :N_NODES, :] + sp_ref[1, :N_NODES, :]
    dv = dinv_ref[...]
    h = dv * s + (dv * dv) * xw2_ref[...] + b2_ref[...]
    h = _bn(_leaky(h), g2_ref[...], be2_ref[...])
    seg = lax.broadcasted_iota(jnp.int32, (N_NODES, NUM_GRAPHS), 1)
    onehot = (seg == batch_ref[...]).astype(jnp.float32)
    pooled = lax.dot_general(onehot, h, (((0,), (0,)), ((), ())),
                             preferred_element_type=jnp.float32)
    z = _leaky(jnp.dot(pooled, w3_ref[...],
                       preferred_element_type=jnp.float32) + b3_ref[...])
    out_ref[...] = jnp.dot(z, w4_ref[...],
                           preferred_element_type=jnp.float32) + b4_ref[...]


def _tc_call(body, out_shapes):
    return pl.pallas_call(body, out_shape=out_shapes)


def kernel(x, edge_index, batch, W1, b1, g1, be1, W2, b2, g2, be2, W3, b3,
           W4, b4):
    row = edge_index[0]
    col = edge_index[1]
    pad = E_PAD - N_EDGES
    row3 = jnp.concatenate([row, jnp.zeros((pad,), jnp.int32)]
                           ).reshape(NW, K, C)
    col3 = jnp.concatenate([col, jnp.full((pad,), DUMMY, jnp.int32)]
                           ).reshape(NW, K, C)
    ones8 = jnp.ones((C, 8), jnp.float32)
    zeros8 = jnp.zeros((R, 8), jnp.float32)
    zeros64 = jnp.zeros((R, H1), jnp.float32)
    zeros32 = jnp.zeros((R, H2), jnp.float32)

    degp = _deg_kernel(col3, ones8, zeros8)

    dinv, xw1, xs1 = _tc_call(
        _tc_a_body,
        [jax.ShapeDtypeStruct((N_NODES, 1), jnp.float32),
         jax.ShapeDtypeStruct((N_NODES, H1), jnp.float32),
         jax.ShapeDtypeStruct((N_NODES, H1), jnp.float32)],
    )(degp, x, W1)

    sp1 = _agg64(row3, col3, xs1, zeros64)

    xw2, xs2 = _tc_call(
        _tc_b_body,
        [jax.ShapeDtypeStruct((N_NODES, H2), jnp.float32),
         jax.ShapeDtypeStruct((N_NODES, H2), jnp.float32)],
    )(sp1, xw1, dinv, b1.reshape(1, H1), g1.reshape(1, H1),
      be1.reshape(1, H1), W2)

    sp2 = _agg32(row3, col3, xs2, zeros32)

    out = _tc_call(
        _tc_c_body,
        jax.ShapeDtypeStruct((NUM_GRAPHS, 1), jnp.float32),
    )(sp2, xw2, dinv, b2.reshape(1, H2), g2.reshape(1, H2),
      be2.reshape(1, H2), batch.reshape(N_NODES, 1), W3,
      b3.reshape(1, MLP1), W4, b4.reshape(1, 1))

    return out


# speedup vs baseline: 46.3974x; 2.8738x over previous
import functools

import jax
import jax.numpy as jnp
from jax import lax
from jax.experimental import pallas as pl
from jax.experimental.pallas import tpu as pltpu
from jax.experimental.pallas import tpu_sc as plsc

N_NODES = 10000
N_EDGES = 320000
IN_CH = 128
H1 = 64
H2 = 32
MLP1 = 16
NUM_GRAPHS = 64

NC = 2
NS = 16
NW = NC * NS
C = 128
K = 80
E_PAD = NW * K * C
R = 10112
STRIPE = R // NS
DUMMY = N_NODES

_mesh = plsc.VectorSubcoreMesh(core_axis_name="c", subcore_axis_name="s")
_sc_params = pltpu.CompilerParams(use_tc_tiling_on_sc=False)


def _worker_id():
    return lax.axis_index("c") * NS + lax.axis_index("s")


def _deg_body(col3, ones_hbm, zeros_hbm, out, idx_c, ones_v, acc):
    cid = lax.axis_index("c")
    sid = lax.axis_index("s")
    wid = _worker_id()
    pltpu.sync_copy(col3.at[wid], idx_c)
    pltpu.sync_copy(ones_hbm, ones_v)
    pltpu.sync_copy(zeros_hbm.at[pl.ds(sid * STRIPE, STRIPE)],
                    acc.at[pl.ds(sid * STRIPE, STRIPE)])
    plsc.subcore_barrier()

    def body(j, carry):
        pltpu.sync_copy(ones_v, acc.at[idx_c.at[j]], add=True)
        return carry

    lax.fori_loop(0, K, body, 0)
    plsc.subcore_barrier()
    pltpu.sync_copy(acc.at[pl.ds(sid * STRIPE, STRIPE)],
                    out.at[cid, pl.ds(sid * STRIPE, STRIPE)])


_deg_kernel = pl.kernel(
    _deg_body,
    out_type=jax.ShapeDtypeStruct((NC, R, 8), jnp.float32),
    mesh=_mesh,
    scratch_types=[
        pltpu.VMEM((K, C), jnp.int32),
        pltpu.VMEM((C, 8), jnp.float32),
        pltpu.VMEM_SHARED((R, 8), jnp.float32),
    ],
    compiler_params=_sc_params,
)


NB = 8


def _agg_body(row3, col3, xs, zeros_hbm, out, idx_r, idx_c, bufs, acc,
              sem_g, sem_s):
    cid = lax.axis_index("c")
    sid = lax.axis_index("s")
    wid = _worker_id()
    pltpu.sync_copy(row3.at[wid], idx_r)
    pltpu.sync_copy(col3.at[wid], idx_c)
    pltpu.sync_copy(zeros_hbm.at[pl.ds(sid * STRIPE, STRIPE)],
                    acc.at[pl.ds(sid * STRIPE, STRIPE)])
    plsc.subcore_barrier()

    G = K // NB

    def wait_gather(b):
        pltpu.make_async_copy(xs.at[idx_r.at[0]], bufs.at[b],
                              sem_g.at[b]).wait()

    def wait_scatter(b):
        pltpu.make_async_copy(bufs.at[b], acc.at[idx_c.at[0]],
                              sem_s.at[b]).wait()

    for b in range(NB):
        pltpu.async_copy(xs.at[idx_r.at[b]], bufs.at[b], sem_g.at[b])

    def body(i, carry):
        for b in range(NB):
            wait_gather(b)
            pltpu.async_copy(bufs.at[b], acc.at[idx_c.at[i * NB + b]],
                             sem_s.at[b], add=True)

        @pl.when(i + 1 < G)
        def _():
            for b in range(NB):
                wait_scatter(b)
                pltpu.async_copy(xs.at[idx_r.at[(i + 1) * NB + b]],
                                 bufs.at[b], sem_g.at[b])

        return carry

    lax.fori_loop(0, G, body, 0)
    for b in range(NB):
        wait_scatter(b)
    plsc.subcore_barrier()
    pltpu.sync_copy(acc.at[pl.ds(sid * STRIPE, STRIPE)],
                    out.at[cid, pl.ds(sid * STRIPE, STRIPE)])


def _make_agg(d):
    return pl.kernel(
        _agg_body,
        out_type=jax.ShapeDtypeStruct((NC, R, d), jnp.float32),
        mesh=_mesh,
        scratch_types=[
            pltpu.VMEM((K, C), jnp.int32),
            pltpu.VMEM((K, C), jnp.int32),
            pltpu.VMEM((NB, C, d), jnp.float32),
            pltpu.VMEM_SHARED((R, d), jnp.float32),
            pltpu.SemaphoreType.DMA((NB,)),
            pltpu.SemaphoreType.DMA((NB,)),
        ],
        compiler_params=_sc_params,
    )


_agg64 = _make_agg(H1)
_agg32 = _make_agg(H2)


def _leaky(v):
    return jnp.where(v >= 0, v, 0.01 * v)


def _bn(h, g, be):
    mu = jnp.mean(h, axis=0, keepdims=True)
    var = jnp.mean((h - mu) * (h - mu), axis=0, keepdims=True)
    return (h - mu) * lax.rsqrt(var + 1e-5) * g + be


def _tc_a_body(degp_ref, x_ref, w1_ref, dinv_ref, xw1_ref, xs1_ref):
    deg = degp_ref[0, :N_NODES, 0:1] + degp_ref[1, :N_NODES, 0:1] + 1.0
    dinv = lax.rsqrt(deg)
    xw = jnp.dot(x_ref[...], w1_ref[...], preferred_element_type=jnp.float32)
    dinv_ref[...] = dinv
    xw1_ref[...] = xw
    xs1_ref[...] = xw * dinv


def _tc_b_body(sp_ref, xw1_ref, dinv_ref, b1_ref, g1_ref, be1_ref, w2_ref,
               xw2_ref, xs2_ref):
    s = sp_ref[0, :N_NODES, :] + sp_ref[1, :N_NODES, :]
    dv = dinv_ref[...]
    xw1 = xw1_ref[...]
    h = dv * s + (dv * dv) * xw1 + b1_ref[...]
    h = _bn(_leaky(h), g1_ref[...], be1_ref[...])
    xw2 = jnp.dot(h, w2_ref[...], preferred_element_type=jnp.float32)
    xw2_ref[...] = xw2
    xs2_ref[...] = xw2 * dv


def _tc_c_body(sp_ref, xw2_ref, dinv_ref, b2_ref, g2_ref, be2_ref, batch_ref,
               w3_ref, b3_ref, w4_ref, b4_ref, out_ref):
    s = sp_ref[0, :N_NODES, :] + sp_ref[1, :N_NODES, :]
    dv = dinv_ref[...]
    h = dv * s + (dv * dv) * xw2_ref[...] + b2_ref[...]
    h = _bn(_leaky(h), g2_ref[...], be2_ref[...])
    seg = lax.broadcasted_iota(jnp.int32, (N_NODES, NUM_GRAPHS), 1)
    onehot = (seg == batch_ref[...]).astype(jnp.float32)
    pooled = lax.dot_general(onehot, h, (((0,), (0,)), ((), ())),
                             preferred_element_type=jnp.float32)
    z = _leaky(jnp.dot(pooled, w3_ref[...],
                       preferred_element_type=jnp.float32) + b3_ref[...])
    out_ref[...] = jnp.dot(z, w4_ref[...],
                           preferred_element_type=jnp.float32) + b4_ref[...]


def _tc_call(body, out_shapes):
    return pl.pallas_call(body, out_shape=out_shapes)


def kernel(x, edge_index, batch, W1, b1, g1, be1, W2, b2, g2, be2, W3, b3,
           W4, b4):
    row = edge_index[0]
    col = edge_index[1]
    pad = E_PAD - N_EDGES
    pad_iota = jnp.arange(pad, dtype=jnp.int32)
    row3 = jnp.concatenate([row, pad_iota % N_NODES]).reshape(NW, K, C)
    col3 = jnp.concatenate([col, DUMMY + pad_iota % (R - N_NODES)]
                           ).reshape(NW, K, C)
    ones8 = jnp.ones((C, 8), jnp.float32)
    zeros8 = jnp.zeros((R, 8), jnp.float32)
    zeros64 = jnp.zeros((R, H1), jnp.float32)
    zeros32 = jnp.zeros((R, H2), jnp.float32)

    degp = _deg_kernel(col3, ones8, zeros8)

    dinv, xw1, xs1 = _tc_call(
        _tc_a_body,
        [jax.ShapeDtypeStruct((N_NODES, 1), jnp.float32),
         jax.ShapeDtypeStruct((N_NODES, H1), jnp.float32),
         jax.ShapeDtypeStruct((N_NODES, H1), jnp.float32)],
    )(degp, x, W1)

    sp1 = _agg64(row3, col3, xs1, zeros64)

    xw2, xs2 = _tc_call(
        _tc_b_body,
        [jax.ShapeDtypeStruct((N_NODES, H2), jnp.float32),
         jax.ShapeDtypeStruct((N_NODES, H2), jnp.float32)],
    )(sp1, xw1, dinv, b1.reshape(1, H1), g1.reshape(1, H1),
      be1.reshape(1, H1), W2)

    sp2 = _agg32(row3, col3, xs2, zeros32)

    out = _tc_call(
        _tc_c_body,
        jax.ShapeDtypeStruct((NUM_GRAPHS, 1), jnp.float32),
    )(sp2, xw2, dinv, b2.reshape(1, H2), g2.reshape(1, H2),
      be2.reshape(1, H2), batch.reshape(N_NODES, 1), W3,
      b3.reshape(1, MLP1), W4, b4.reshape(1, 1))

    return out


# speedup vs baseline: 47.6820x; 1.0277x over previous
import functools

import jax
import jax.numpy as jnp
from jax import lax
from jax.experimental import pallas as pl
from jax.experimental.pallas import tpu as pltpu
from jax.experimental.pallas import tpu_sc as plsc

N_NODES = 10000
N_EDGES = 320000
IN_CH = 128
H1 = 64
H2 = 32
MLP1 = 16
NUM_GRAPHS = 64

NC = 2
NS = 16
NW = NC * NS
C = 128
K = 80
E_PAD = NW * K * C
R = 10112
STRIPE = R // NS
DUMMY = N_NODES

_mesh = plsc.VectorSubcoreMesh(core_axis_name="c", subcore_axis_name="s")
_sc_params = pltpu.CompilerParams(use_tc_tiling_on_sc=False)


def _worker_id():
    return lax.axis_index("c") * NS + lax.axis_index("s")


def _deg_body(col3, ones_hbm, zeros_hbm, out, idx_c, ones_v, acc, sem):
    cid = lax.axis_index("c")
    sid = lax.axis_index("s")
    wid = _worker_id()
    pltpu.sync_copy(col3.at[wid], idx_c)
    pltpu.sync_copy(ones_hbm, ones_v)
    pltpu.sync_copy(zeros_hbm.at[pl.ds(sid * STRIPE, STRIPE)],
                    acc.at[pl.ds(sid * STRIPE, STRIPE)])
    plsc.subcore_barrier()

    def body(j, carry):
        pltpu.async_copy(ones_v, acc.at[idx_c.at[j]], sem, add=True)
        return carry

    lax.fori_loop(0, K, body, 0)

    def drain(j, carry):
        pltpu.make_async_copy(ones_v, acc.at[idx_c.at[0]], sem).wait()
        return carry

    lax.fori_loop(0, K, drain, 0)
    plsc.subcore_barrier()
    pltpu.sync_copy(acc.at[pl.ds(sid * STRIPE, STRIPE)],
                    out.at[cid, pl.ds(sid * STRIPE, STRIPE)])


_deg_kernel = pl.kernel(
    _deg_body,
    out_type=jax.ShapeDtypeStruct((NC, R, 8), jnp.float32),
    mesh=_mesh,
    scratch_types=[
        pltpu.VMEM((K, C), jnp.int32),
        pltpu.VMEM((C, 8), jnp.float32),
        pltpu.VMEM_SHARED((R, 8), jnp.float32),
        pltpu.SemaphoreType.DMA,
    ],
    compiler_params=_sc_params,
)


NB = 8


def _agg_body(row3, col3, xs, zeros_hbm, out, idx_r, idx_c, bufs, acc,
              sem_g, sem_s):
    cid = lax.axis_index("c")
    sid = lax.axis_index("s")
    wid = _worker_id()
    pltpu.sync_copy(row3.at[wid], idx_r)
    pltpu.sync_copy(col3.at[wid], idx_c)
    pltpu.sync_copy(zeros_hbm.at[pl.ds(sid * STRIPE, STRIPE)],
                    acc.at[pl.ds(sid * STRIPE, STRIPE)])
    plsc.subcore_barrier()

    G = K // NB

    def wait_gather(b):
        pltpu.make_async_copy(xs.at[idx_r.at[0]], bufs.at[b],
                              sem_g.at[b]).wait()

    def wait_scatter(b):
        pltpu.make_async_copy(bufs.at[b], acc.at[idx_c.at[0]],
                              sem_s.at[b]).wait()

    for b in range(NB):
        pltpu.async_copy(xs.at[idx_r.at[b]], bufs.at[b], sem_g.at[b])

    def body(i, carry):
        for b in range(NB):
            wait_gather(b)
            pltpu.async_copy(bufs.at[b], acc.at[idx_c.at[i * NB + b]],
                             sem_s.at[b], add=True)

        @pl.when(i + 1 < G)
        def _():
            for b in range(NB):
                wait_scatter(b)
                pltpu.async_copy(xs.at[idx_r.at[(i + 1) * NB + b]],
                                 bufs.at[b], sem_g.at[b])

        return carry

    lax.fori_loop(0, G, body, 0)
    for b in range(NB):
        wait_scatter(b)
    plsc.subcore_barrier()
    pltpu.sync_copy(acc.at[pl.ds(sid * STRIPE, STRIPE)],
                    out.at[cid, pl.ds(sid * STRIPE, STRIPE)])


def _make_agg(d):
    return pl.kernel(
        _agg_body,
        out_type=jax.ShapeDtypeStruct((NC, R, d), jnp.float32),
        mesh=_mesh,
        scratch_types=[
            pltpu.VMEM((K, C), jnp.int32),
            pltpu.VMEM((K, C), jnp.int32),
            pltpu.VMEM((NB, C, d), jnp.float32),
            pltpu.VMEM_SHARED((R, d), jnp.float32),
            pltpu.SemaphoreType.DMA((NB,)),
            pltpu.SemaphoreType.DMA((NB,)),
        ],
        compiler_params=_sc_params,
    )


_agg64 = _make_agg(H1)
_agg32 = _make_agg(H2)


def _leaky(v):
    return jnp.where(v >= 0, v, 0.01 * v)


def _bn(h, g, be):
    mu = jnp.mean(h, axis=0, keepdims=True)
    var = jnp.mean((h - mu) * (h - mu), axis=0, keepdims=True)
    return (h - mu) * lax.rsqrt(var + 1e-5) * g + be


def _tc_mm_body(x_ref, w1_ref, xw1_ref):
    xw1_ref[...] = jnp.dot(x_ref[...], w1_ref[...],
                           preferred_element_type=jnp.float32)


def _tc_a_body(degp_ref, xw_ref, dinv_ref, xs1_ref):
    deg = degp_ref[0, :N_NODES, 0:1] + degp_ref[1, :N_NODES, 0:1] + 1.0
    dinv = lax.rsqrt(deg)
    dinv_ref[...] = dinv
    xs1_ref[...] = xw_ref[...] * dinv


def _tc_b_body(sp_ref, xw1_ref, dinv_ref, b1_ref, g1_ref, be1_ref, w2_ref,
               xw2_ref, xs2_ref):
    s = sp_ref[0, :N_NODES, :] + sp_ref[1, :N_NODES, :]
    dv = dinv_ref[...]
    xw1 = xw1_ref[...]
    h = dv * s + (dv * dv) * xw1 + b1_ref[...]
    h = _bn(_leaky(h), g1_ref[...], be1_ref[...])
    xw2 = jnp.dot(h, w2_ref[...], preferred_element_type=jnp.float32)
    xw2_ref[...] = xw2
    xs2_ref[...] = xw2 * dv


def _tc_c_body(sp_ref, xw2_ref, dinv_ref, b2_ref, g2_ref, be2_ref, batch_ref,
               w3_ref, b3_ref, w4_ref, b4_ref, out_ref):
    s = sp_ref[0, :N_NODES, :] + sp_ref[1, :N_NODES, :]
    dv = dinv_ref[...]
    h = dv * s + (dv * dv) * xw2_ref[...] + b2_ref[...]
    h = _bn(_leaky(h), g2_ref[...], be2_ref[...])
    seg = lax.broadcasted_iota(jnp.int32, (N_NODES, NUM_GRAPHS), 1)
    onehot = (seg == batch_ref[...]).astype(jnp.float32)
    pooled = lax.dot_general(onehot, h, (((0,), (0,)), ((), ())),
                             preferred_element_type=jnp.float32)
    z = _leaky(jnp.dot(pooled, w3_ref[...],
                       preferred_element_type=jnp.float32) + b3_ref[...])
    out_ref[...] = jnp.dot(z, w4_ref[...],
                           preferred_element_type=jnp.float32) + b4_ref[...]


def _tc_call(body, out_shapes):
    return pl.pallas_call(body, out_shape=out_shapes)


def kernel(x, edge_index, batch, W1, b1, g1, be1, W2, b2, g2, be2, W3, b3,
           W4, b4):
    row = edge_index[0]
    col = edge_index[1]
    pad = E_PAD - N_EDGES
    pad_iota = jnp.arange(pad, dtype=jnp.int32)
    row3 = jnp.concatenate([row, pad_iota % N_NODES]).reshape(NW, K, C)
    col3 = jnp.concatenate([col, DUMMY + pad_iota % (R - N_NODES)]
                           ).reshape(NW, K, C)
    ones8 = jnp.ones((C, 8), jnp.float32)
    zeros8 = jnp.zeros((R, 8), jnp.float32)
    zeros64 = jnp.zeros((R, H1), jnp.float32)
    zeros32 = jnp.zeros((R, H2), jnp.float32)

    xw1 = _tc_call(
        _tc_mm_body,
        jax.ShapeDtypeStruct((N_NODES, H1), jnp.float32),
    )(x, W1)

    degp = _deg_kernel(col3, ones8, zeros8)

    dinv, xs1 = _tc_call(
        _tc_a_body,
        [jax.ShapeDtypeStruct((N_NODES, 1), jnp.float32),
         jax.ShapeDtypeStruct((N_NODES, H1), jnp.float32)],
    )(degp, xw1)

    sp1 = _agg64(row3, col3, xs1, zeros64)

    xw2, xs2 = _tc_call(
        _tc_b_body,
        [jax.ShapeDtypeStruct((N_NODES, H2), jnp.float32),
         jax.ShapeDtypeStruct((N_NODES, H2), jnp.float32)],
    )(sp1, xw1, dinv, b1.reshape(1, H1), g1.reshape(1, H1),
      be1.reshape(1, H1), W2)

    sp2 = _agg32(row3, col3, xs2, zeros32)

    out = _tc_call(
        _tc_c_body,
        jax.ShapeDtypeStruct((NUM_GRAPHS, 1), jnp.float32),
    )(sp2, xw2, dinv, b2.reshape(1, H2), g2.reshape(1, H2),
      be2.reshape(1, H2), batch.reshape(N_NODES, 1), W3,
      b3.reshape(1, MLP1), W4, b4.reshape(1, 1))

    return out


# speedup vs baseline: 49.4786x; 1.0377x over previous
import functools

import jax
import jax.numpy as jnp
from jax import lax
from jax.experimental import pallas as pl
from jax.experimental.pallas import tpu as pltpu
from jax.experimental.pallas import tpu_sc as plsc

N_NODES = 10000
N_EDGES = 320000
IN_CH = 128
H1 = 64
H2 = 32
MLP1 = 16
NUM_GRAPHS = 64

NC = 2
NS = 16
NW = NC * NS
C = 128
K = 80
E_PAD = NW * K * C
R = 10112
STRIPE = R // NS
DUMMY = N_NODES

_mesh = plsc.VectorSubcoreMesh(core_axis_name="c", subcore_axis_name="s")
_sc_params = pltpu.CompilerParams(use_tc_tiling_on_sc=False)


def _worker_id():
    return lax.axis_index("c") * NS + lax.axis_index("s")


def _deg_body(col3, ones_hbm, zeros_hbm, out, idx_c, ones_v, acc, sem):
    cid = lax.axis_index("c")
    sid = lax.axis_index("s")
    wid = _worker_id()
    pltpu.sync_copy(col3.at[wid], idx_c)
    pltpu.sync_copy(ones_hbm, ones_v)
    pltpu.sync_copy(zeros_hbm.at[pl.ds(sid * STRIPE, STRIPE)],
                    acc.at[pl.ds(sid * STRIPE, STRIPE)])
    plsc.subcore_barrier()

    def body(j, carry):
        pltpu.async_copy(ones_v, acc.at[idx_c.at[j]], sem, add=True)
        return carry

    lax.fori_loop(0, K, body, 0)

    def drain(j, carry):
        pltpu.make_async_copy(ones_v, acc.at[idx_c.at[0]], sem).wait()
        return carry

    lax.fori_loop(0, K, drain, 0)
    plsc.subcore_barrier()
    pltpu.sync_copy(acc.at[pl.ds(sid * STRIPE, STRIPE)],
                    out.at[cid, pl.ds(sid * STRIPE, STRIPE)])


_deg_kernel = pl.kernel(
    _deg_body,
    out_type=jax.ShapeDtypeStruct((NC, R, 8), jnp.float32),
    mesh=_mesh,
    scratch_types=[
        pltpu.VMEM((K, C), jnp.int32),
        pltpu.VMEM((C, 8), jnp.float32),
        pltpu.VMEM_SHARED((R, 8), jnp.float32),
        pltpu.SemaphoreType.DMA,
    ],
    compiler_params=_sc_params,
)


NB = 8


def _agg_body(row3, col3, xs, zeros_hbm, out, idx_r, idx_c, bufs, acc,
              sem_g, sem_s):
    cid = lax.axis_index("c")
    sid = lax.axis_index("s")
    wid = _worker_id()
    pltpu.sync_copy(row3.at[wid], idx_r)
    pltpu.sync_copy(col3.at[wid], idx_c)

    G = K // NB

    def wait_gather(b):
        pltpu.make_async_copy(xs.at[idx_r.at[0]], bufs.at[b],
                              sem_g.at[b]).wait()

    def wait_scatter(b):
        pltpu.make_async_copy(bufs.at[b], acc.at[idx_c.at[0]],
                              sem_s.at[b]).wait()

    for b in range(NB):
        pltpu.async_copy(xs.at[idx_r.at[b]], bufs.at[b], sem_g.at[b])
    pltpu.sync_copy(zeros_hbm.at[pl.ds(sid * STRIPE, STRIPE)],
                    acc.at[pl.ds(sid * STRIPE, STRIPE)])
    plsc.subcore_barrier()

    def body(i, carry):
        for b in range(NB):
            wait_gather(b)
            pltpu.async_copy(bufs.at[b], acc.at[idx_c.at[i * NB + b]],
                             sem_s.at[b], add=True)

        @pl.when(i + 1 < G)
        def _():
            for b in range(NB):
                wait_scatter(b)
                pltpu.async_copy(xs.at[idx_r.at[(i + 1) * NB + b]],
                                 bufs.at[b], sem_g.at[b])

        return carry

    lax.fori_loop(0, G, body, 0)
    for b in range(NB):
        wait_scatter(b)
    plsc.subcore_barrier()
    pltpu.sync_copy(acc.at[pl.ds(sid * STRIPE, STRIPE)],
                    out.at[cid, pl.ds(sid * STRIPE, STRIPE)])


def _make_agg(d):
    return pl.kernel(
        _agg_body,
        out_type=jax.ShapeDtypeStruct((NC, R, d), jnp.float32),
        mesh=_mesh,
        scratch_types=[
            pltpu.VMEM((K, C), jnp.int32),
            pltpu.VMEM((K, C), jnp.int32),
            pltpu.VMEM((NB, C, d), jnp.float32),
            pltpu.VMEM_SHARED((R, d), jnp.float32),
            pltpu.SemaphoreType.DMA((NB,)),
            pltpu.SemaphoreType.DMA((NB,)),
        ],
        compiler_params=_sc_params,
    )


_agg64 = _make_agg(H1)
_agg32 = _make_agg(H2)


def _leaky(v):
    return jnp.where(v >= 0, v, 0.01 * v)


def _bn(h, g, be):
    mu = jnp.mean(h, axis=0, keepdims=True)
    var = jnp.mean((h - mu) * (h - mu), axis=0, keepdims=True)
    return (h - mu) * lax.rsqrt(var + 1e-5) * g + be


def _tc_a_body(degp_ref, x_ref, w1_ref, dinv_ref, xs1_ref):
    deg = degp_ref[0, :N_NODES, 0:1] + degp_ref[1, :N_NODES, 0:1] + 1.0
    dinv = lax.rsqrt(deg)
    dinv_ref[...] = dinv
    xs1_ref[...] = jnp.dot(x_ref[...] * dinv, w1_ref[...],
                           preferred_element_type=jnp.float32)


def _tc_b_body(sp_ref, xs1_ref, dinv_ref, b1_ref, g1_ref, be1_ref, w2_ref,
               xs2_ref):
    s = sp_ref[0, :N_NODES, :] + sp_ref[1, :N_NODES, :]
    dv = dinv_ref[...]
    h = dv * (s + xs1_ref[...]) + b1_ref[...]
    h = _bn(_leaky(h), g1_ref[...], be1_ref[...])
    xs2_ref[...] = jnp.dot(h * dv, w2_ref[...],
                           preferred_element_type=jnp.float32)


def _tc_c_body(sp_ref, xs2_ref, dinv_ref, b2_ref, g2_ref, be2_ref, batch_ref,
               w3_ref, b3_ref, w4_ref, b4_ref, out_ref):
    s = sp_ref[0, :N_NODES, :] + sp_ref[1, :N_NODES, :]
    dv = dinv_ref[...]
    h = dv * (s + xs2_ref[...]) + b2_ref[...]
    h = _bn(_leaky(h), g2_ref[...], be2_ref[...])
    seg = lax.broadcasted_iota(jnp.int32, (N_NODES, NUM_GRAPHS), 1)
    onehot = (seg == batch_ref[...]).astype(jnp.float32)
    pooled = lax.dot_general(onehot, h, (((0,), (0,)), ((), ())),
                             preferred_element_type=jnp.float32)
    z = _leaky(jnp.dot(pooled, w3_ref[...],
                       preferred_element_type=jnp.float32) + b3_ref[...])
    out_ref[...] = jnp.dot(z, w4_ref[...],
                           preferred_element_type=jnp.float32) + b4_ref[...]


def _tc_call(body, out_shapes):
    return pl.pallas_call(body, out_shape=out_shapes)


def kernel(x, edge_index, batch, W1, b1, g1, be1, W2, b2, g2, be2, W3, b3,
           W4, b4):
    row = edge_index[0]
    col = edge_index[1]
    pad = E_PAD - N_EDGES
    pad_iota = jnp.arange(pad, dtype=jnp.int32)
    row3 = jnp.concatenate([row, pad_iota % N_NODES]).reshape(NW, K, C)
    col3 = jnp.concatenate([col, DUMMY + pad_iota % (R - N_NODES)]
                           ).reshape(NW, K, C)
    ones8 = jnp.ones((C, 8), jnp.float32)
    zeros8 = jnp.zeros((R, 8), jnp.float32)
    zeros64 = jnp.zeros((R, H1), jnp.float32)
    zeros32 = jnp.zeros((R, H2), jnp.float32)

    degp = _deg_kernel(col3, ones8, zeros8)

    dinv, xs1 = _tc_call(
        _tc_a_body,
        [jax.ShapeDtypeStruct((N_NODES, 1), jnp.float32),
         jax.ShapeDtypeStruct((N_NODES, H1), jnp.float32)],
    )(degp, x, W1)

    sp1 = _agg64(row3, col3, xs1, zeros64)

    xs2 = _tc_call(
        _tc_b_body,
        jax.ShapeDtypeStruct((N_NODES, H2), jnp.float32),
    )(sp1, xs1, dinv, b1.reshape(1, H1), g1.reshape(1, H1),
      be1.reshape(1, H1), W2)

    sp2 = _agg32(row3, col3, xs2, zeros32)

    out = _tc_call(
        _tc_c_body,
        jax.ShapeDtypeStruct((NUM_GRAPHS, 1), jnp.float32),
    )(sp2, xs2, dinv, b2.reshape(1, H2), g2.reshape(1, H2),
      be2.reshape(1, H2), batch.reshape(N_NODES, 1), W3,
      b3.reshape(1, MLP1), W4, b4.reshape(1, 1))

    return out
